# Initial kernel scaffold; baseline (speedup 1.0000x reference)
import functools

import jax
import jax.numpy as jnp
import numpy as np
from jax.experimental import pallas as pl
from jax.experimental.pallas import tpu as pltpu

BS = 8
NQ = 900
NC = 92
TPI = 25
TT = BS * TPI
PAD = 32
PCOLS = BS * PAD

_SQRT2 = np.float32(np.sqrt(np.float32(2.0)))
_DEN = np.float32(2.5) * _SQRT2
_HN = np.float32(0.3) * _SQRT2
_NEG = np.float32(-3.0e38)


def _kernel(logits_ref, pb_ref, tb_ref, lab_ref, out_ref, B_ref):
    for b in range(BS):
        x = logits_ref[b]
        m = jnp.max(x, axis=1, keepdims=True)
        e = jnp.exp(x - m)
        s = jnp.sum(e, axis=1, keepdims=True)
        prob = e / s

        lab = lab_ref[b]
        onehot = (lab == jax.lax.broadcasted_iota(jnp.int32, (PAD, NC), 1)
                  ).astype(jnp.float32)
        clsprob = jax.lax.dot_general(
            onehot, prob, (((1,), (1,)), ((), ())),
            preferred_element_type=jnp.float32)

        pb = pb_ref[b]
        qcx, qcy, qw, qh = pb[0:1], pb[1:2], pb[2:3], pb[3:4]
        tb = tb_ref[b]
        tcx, tcy, tw, th = tb[:, 0:1], tb[:, 1:2], tb[:, 2:3], tb[:, 3:4]

        cbbox = (jnp.abs(qcx - tcx) + jnp.abs(qcy - tcy)
                 + jnp.abs(qw - tw) + jnp.abs(qh - th))

        qx0, qx1 = qcx - 0.5 * qw, qcx + 0.5 * qw
        qy0, qy1 = qcy - 0.5 * qh, qcy + 0.5 * qh
        tx0, tx1 = tcx - 0.5 * tw, tcx + 0.5 * tw
        ty0, ty1 = tcy - 0.5 * th, tcy + 0.5 * th

        area_q = (qx1 - qx0) * (qy1 - qy0)
        area_t = (tx1 - tx0) * (ty1 - ty0)
        ltx = jnp.maximum(qx0, tx0)
        lty = jnp.maximum(qy0, ty0)
        rbx = jnp.minimum(qx1, tx1)
        rby = jnp.minimum(qy1, ty1)
        iw = jnp.clip(rbx - ltx, 0.0, None)
        ih = jnp.clip(rby - lty, 0.0, None)
        inter = iw * ih
        union = area_q + area_t - inter
        iou = inter / union
        l2x = jnp.minimum(qx0, tx0)
        l2y = jnp.minimum(qy0, ty0)
        r2x = jnp.maximum(qx1, tx1)
        r2y = jnp.maximum(qy1, ty1)
        ew = jnp.clip(r2x - l2x, 0.0, None)
        eh = jnp.clip(r2y - l2y, 0.0, None)
        earea = ew * eh
        giou = iou - (earea - union) / earea

        C = 5.0 * cbbox + 1.0 * (-clsprob) + 2.0 * (-giou)

        z = (-C - (-5.5)) / _DEN
        p = 0.5 * (1.0 + jax.lax.erf(z))
        fxa = _HN * jax.lax.erf_inv(p)
        B_ref[PAD * b:PAD * (b + 1), :] = fxa

    col_iota = jax.lax.broadcasted_iota(jnp.int32, (PCOLS, 1), 0)
    q_iota = jax.lax.broadcasted_iota(jnp.int32, (PCOLS, NQ), 1)
    pad_col = (col_iota % PAD) >= TPI

    def body(_, carry):
        B = B_ref[...]
        Bc = jnp.minimum(B, 100.0)
        max1 = jnp.max(Bc, axis=1, keepdims=True)
        amax_c = jnp.min(jnp.where(Bc == max1, q_iota, NQ),
                         axis=1, keepdims=True)
        second = jnp.max(jnp.where(q_iota == amax_c, _NEG, Bc),
                         axis=1, keepdims=True)
        maxu = jnp.max(B, axis=1, keepdims=True)
        amax_u = jnp.min(jnp.where(B == maxu, q_iota, NQ),
                         axis=1, keepdims=True)
        gap = jnp.where(pad_col, -1.0, max1 - second)

        gmax = jnp.max(gap)
        col = jnp.min(jnp.where(gap == gmax, col_iota, PCOLS))
        row = jnp.min(jnp.where(col_iota == col, amax_u, NQ))
        img = col // PAD

        colmask = col_iota == col
        imgmask = (col_iota // PAD) == img
        rowmask = q_iota == row
        B = jnp.where(colmask & rowmask, np.float32(-1e-7),
                      jnp.where(colmask | (imgmask & rowmask), 0.0, B))
        B_ref[...] = B
        return carry

    jax.lax.fori_loop(0, TT, body, jnp.int32(0))
    out_ref[...] = (B_ref[...] < 0.0).astype(jnp.uint8)


@jax.jit
def kernel(pred_logits, pred_boxes, tgt_labels, tgt_boxes):
    pb = jnp.transpose(pred_boxes, (0, 2, 1))
    tb = tgt_boxes.reshape(BS, TPI, 4)
    tb = jnp.pad(tb, ((0, 0), (0, PAD - TPI), (0, 0)))
    lab = tgt_labels.reshape(BS, TPI)
    lab = jnp.pad(lab, ((0, 0), (0, PAD - TPI)))[..., None]

    out = pl.pallas_call(
        _kernel,
        out_shape=jax.ShapeDtypeStruct((PCOLS, NQ), jnp.uint8),
        scratch_shapes=[pltpu.VMEM((PCOLS, NQ), jnp.float32)],
    )(pred_logits, pb, tb, lab)

    sel = (np.arange(TT) // TPI) * PAD + (np.arange(TT) % TPI)
    return out[sel, :].T.astype(jnp.bool_)


# speedup vs baseline: 96.4172x; 96.4172x over previous
import functools

import jax
import jax.numpy as jnp
import numpy as np
from jax.experimental import pallas as pl
from jax.experimental.pallas import tpu as pltpu

BS = 8
NQ = 900
NC = 92
TPI = 25
TT = BS * TPI
PAD = 32
PCOLS = BS * PAD

_SQRT2 = np.float32(np.sqrt(np.float32(2.0)))
_DEN = np.float32(2.5) * _SQRT2
_HN = np.float32(0.3) * _SQRT2
_NEG = np.float32(-3.0e38)


def _kernel(logits_ref, pb_ref, tb_ref, lab_ref, out_ref, B_ref):
    for b in range(BS):
        x = logits_ref[b]
        m = jnp.max(x, axis=1, keepdims=True)
        e = jnp.exp(x - m)
        s = jnp.sum(e, axis=1, keepdims=True)
        prob = e / s

        lab = lab_ref[b]
        onehot = (lab == jax.lax.broadcasted_iota(jnp.int32, (PAD, NC), 1)
                  ).astype(jnp.float32)
        clsprob = jax.lax.dot_general(
            onehot, prob, (((1,), (1,)), ((), ())),
            preferred_element_type=jnp.float32)

        pb = pb_ref[b]
        qcx, qcy, qw, qh = pb[0:1], pb[1:2], pb[2:3], pb[3:4]
        tb = tb_ref[b]
        tcx, tcy, tw, th = tb[:, 0:1], tb[:, 1:2], tb[:, 2:3], tb[:, 3:4]

        cbbox = (jnp.abs(qcx - tcx) + jnp.abs(qcy - tcy)
                 + jnp.abs(qw - tw) + jnp.abs(qh - th))

        qx0, qx1 = qcx - 0.5 * qw, qcx + 0.5 * qw
        qy0, qy1 = qcy - 0.5 * qh, qcy + 0.5 * qh
        tx0, tx1 = tcx - 0.5 * tw, tcx + 0.5 * tw
        ty0, ty1 = tcy - 0.5 * th, tcy + 0.5 * th

        area_q = (qx1 - qx0) * (qy1 - qy0)
        area_t = (tx1 - tx0) * (ty1 - ty0)
        ltx = jnp.maximum(qx0, tx0)
        lty = jnp.maximum(qy0, ty0)
        rbx = jnp.minimum(qx1, tx1)
        rby = jnp.minimum(qy1, ty1)
        iw = jnp.clip(rbx - ltx, 0.0, None)
        ih = jnp.clip(rby - lty, 0.0, None)
        inter = iw * ih
        union = area_q + area_t - inter
        iou = inter / union
        l2x = jnp.minimum(qx0, tx0)
        l2y = jnp.minimum(qy0, ty0)
        r2x = jnp.maximum(qx1, tx1)
        r2y = jnp.maximum(qy1, ty1)
        ew = jnp.clip(r2x - l2x, 0.0, None)
        eh = jnp.clip(r2y - l2y, 0.0, None)
        earea = ew * eh
        giou = iou - (earea - union) / earea

        C = 5.0 * cbbox + 1.0 * (-clsprob) + 2.0 * (-giou)

        z = (-C - (-5.5)) / _DEN
        p = 0.5 * (1.0 + jax.lax.erf(z))
        fxa = _HN * jax.lax.erf_inv(p)
        B_ref[PAD * b:PAD * (b + 1), :] = fxa

    col_iota = jax.lax.broadcasted_iota(jnp.int32, (PCOLS, 1), 0)
    q_iota = jax.lax.broadcasted_iota(jnp.int32, (PCOLS, NQ), 1)
    pad_col = (col_iota % PAD) >= TPI

    def body(_, carry):
        B = B_ref[...]
        Bc = jnp.minimum(B, 100.0)
        max1 = jnp.max(Bc, axis=1, keepdims=True)
        amax_c = jnp.min(jnp.where(Bc == max1, q_iota, NQ),
                         axis=1, keepdims=True)
        second = jnp.max(jnp.where(q_iota == amax_c, _NEG, Bc),
                         axis=1, keepdims=True)
        maxu = jnp.max(B, axis=1, keepdims=True)
        amax_u = jnp.min(jnp.where(B == maxu, q_iota, NQ),
                         axis=1, keepdims=True)
        gap = jnp.where(pad_col, -1.0, max1 - second)

        gmax = jnp.max(gap)
        col = jnp.min(jnp.where(gap == gmax, col_iota, PCOLS))
        row = jnp.min(jnp.where(col_iota == col, amax_u, NQ))

        colmask = col_iota == col
        col01 = col_iota <= 1
        rowmask = q_iota == row
        B = jnp.where(colmask & rowmask, np.float32(-1e-7),
                      jnp.where(colmask | (col01 & rowmask), 0.0, B))
        B_ref[...] = B
        return carry

    jax.lax.fori_loop(0, TT, body, jnp.int32(0))
    out_ref[...] = (B_ref[...] < 0.0).astype(jnp.uint8)


@jax.jit
def kernel(pred_logits, pred_boxes, tgt_labels, tgt_boxes):
    pb = jnp.transpose(pred_boxes, (0, 2, 1))
    tb = tgt_boxes.reshape(BS, TPI, 4)
    tb = jnp.pad(tb, ((0, 0), (0, PAD - TPI), (0, 0)))
    lab = tgt_labels.reshape(BS, TPI)
    lab = jnp.pad(lab, ((0, 0), (0, PAD - TPI)))[..., None]

    out = pl.pallas_call(
        _kernel,
        out_shape=jax.ShapeDtypeStruct((PCOLS, NQ), jnp.uint8),
        scratch_shapes=[pltpu.VMEM((PCOLS, NQ), jnp.float32)],
    )(pred_logits, pb, tb, lab)

    sel = (np.arange(TT) // TPI) * PAD + (np.arange(TT) % TPI)
    return out[sel, :].T.astype(jnp.bool_)
